# Initial kernel scaffold; baseline (speedup 1.0000x reference)
import numpy as np
import jax
import jax.numpy as jnp
from jax.experimental import pallas as pl
from jax.experimental.pallas import tpu as pltpu

H, W = 5000, 4000
h, w = 1000, 800
RB = 500
NB = H // RB
ORS = RB // 5
KSLOTS = 104


def _build_row_weights() -> np.ndarray:
    m = np.zeros((KSLOTS, RB), np.float32)
    for k in range(ORS + 2):
        c = 5 * k - 3
        for l in range(max(0, c - 4), min(RB - 1, c + 4) + 1):
            m[k, l] = (1.0 - abs(l - c) / 5.0) / 5.0
    return m


def _build_col_weights() -> np.ndarray:
    m = np.zeros((W, w), np.float32)
    for c in range(w):
        ctr = 5 * c + 2
        for j in range(max(0, ctr - 4), min(W - 1, ctr + 4) + 1):
            m[j, c] = (1.0 - abs(j - ctr) / 5.0) / 5.0
    return m


_WR = _build_row_weights()
_WC = _build_col_weights()


def _downsample_kernel(x_ref, wr_ref, wc_ref, acc_ref):
    b = pl.program_id(0)

    @pl.when(b == 0)
    def _():
        acc_ref[...] = jnp.zeros_like(acc_ref)

    xb = x_ref[...].astype(jnp.bfloat16)
    t1 = jax.lax.dot_general(
        wr_ref[...], xb, (((1,), (0,)), ((), ())),
        preferred_element_type=jnp.float32)
    t2 = jax.lax.dot_general(
        t1.astype(jnp.bfloat16), wc_ref[...], (((1,), (0,)), ((), ())),
        preferred_element_type=jnp.float32)
    acc_ref[pl.ds(ORS * b, KSLOTS), :] += t2


def _slide13(v, combine, axis):
    n = v.shape[axis] - 24

    def sl(a, start, size):
        idx = [slice(None)] * a.ndim
        idx[axis] = slice(start, start + size)
        return a[tuple(idx)]

    a1 = combine(sl(v, 0, n + 23), sl(v, 1, n + 23))
    a2 = combine(sl(a1, 0, n + 21), sl(a1, 2, n + 21))
    a4 = combine(sl(a2, 0, n + 17), sl(a2, 4, n + 17))
    return combine(combine(sl(a4, 2, n), sl(a2, 10, n)), sl(v, 14, n))


def _morph_open_13(x1):
    zr = jnp.zeros((8, w), jnp.float32)
    zr2 = jnp.zeros((16, w), jnp.float32)
    zc = jnp.zeros((h, 8), jnp.float32)
    zc2 = jnp.zeros((h, 16), jnp.float32)

    fr = jnp.concatenate([zr, x1, zr2], axis=0)
    er_r = _slide13(fr, jnp.minimum, 0)
    fc = jnp.concatenate([zc, er_r, zc2], axis=1)
    er = _slide13(fc, jnp.minimum, 1)

    gr = jnp.concatenate([zr, er, zr2], axis=0)
    di_r = _slide13(gr, jnp.maximum, 0)
    gc = jnp.concatenate([zc, di_r, zc2], axis=1)
    return _slide13(gc, jnp.maximum, 1)


def _mask_kernel(acc_ref, rm_ref, cm_ref):
    x1 = acc_ref[...][1:h + 1, :]
    di = _morph_open_13(x1)
    p = jnp.where(di > 0.0, 1.0, 0.0).astype(jnp.float32)

    y_cnt = jnp.sum(p, axis=1, keepdims=True)
    P = jnp.where(y_cnt > 0.0, 1.0, 0.0).astype(jnp.float32)
    Pprev = jnp.concatenate([P[0:1], P[0:h - 1]], axis=0)
    Pnext = jnp.concatenate([P[1:h], P[h - 1:h]], axis=0)
    m01 = jnp.maximum(Pprev, P)
    m234 = jnp.maximum(P, Pnext)
    lane = jax.lax.broadcasted_iota(jnp.int32, (h, 8), 1)
    rm_ref[...] = jnp.where(lane < 2, m01, m234)

    x_cnt = jnp.sum(p, axis=0, keepdims=True)
    Q = jnp.where(x_cnt > 0.0, 1.0, 0.0).astype(jnp.float32)
    Qprev = jnp.concatenate([Q[:, 0:1], Q[:, 0:w - 1]], axis=1)
    Qnext = jnp.concatenate([Q[:, 1:w], Q[:, w - 1:w]], axis=1)
    n01 = jnp.maximum(Qprev, Q)
    n234 = jnp.maximum(Q, Qnext)
    sub = jax.lax.broadcasted_iota(jnp.int32, (8, w), 0)
    cm_ref[...] = jnp.where(sub < 2, n01, n234)


def _apply_kernel(x_ref, rm_ref, cm_ref, o_ref):
    o_ref[...] = x_ref[...] * rm_ref[...] * cm_ref[...]


def kernel(x):
    wr = jnp.asarray(_WR, dtype=jnp.bfloat16)
    wc = jnp.asarray(_WC, dtype=jnp.bfloat16)

    acc = pl.pallas_call(
        _downsample_kernel,
        grid=(NB,),
        in_specs=[
            pl.BlockSpec((RB, W), lambda b: (b, 0)),
            pl.BlockSpec((KSLOTS, RB), lambda b: (0, 0)),
            pl.BlockSpec((W, w), lambda b: (0, 0)),
        ],
        out_specs=pl.BlockSpec((h + 8, w), lambda b: (0, 0)),
        out_shape=jax.ShapeDtypeStruct((h + 8, w), jnp.float32),
        compiler_params=pltpu.CompilerParams(
            dimension_semantics=("arbitrary",),
            vmem_limit_bytes=48 * 1024 * 1024,
        ),
        name="filter_downsample",
    )(x, wr, wc)

    rm5, cm5 = pl.pallas_call(
        _mask_kernel,
        out_shape=[
            jax.ShapeDtypeStruct((h, 8), jnp.float32),
            jax.ShapeDtypeStruct((8, w), jnp.float32),
        ],
        compiler_params=pltpu.CompilerParams(
            vmem_limit_bytes=52 * 1024 * 1024,
        ),
        name="filter_mask",
    )(acc)

    rm = rm5[:, :5].reshape(H, 1)
    cm = cm5[:5, :].T.reshape(1, W)

    return pl.pallas_call(
        _apply_kernel,
        grid=(NB,),
        in_specs=[
            pl.BlockSpec((RB, W), lambda b: (b, 0)),
            pl.BlockSpec((RB, 1), lambda b: (b, 0)),
            pl.BlockSpec((1, W), lambda b: (0, 0)),
        ],
        out_specs=pl.BlockSpec((RB, W), lambda b: (b, 0)),
        out_shape=jax.ShapeDtypeStruct((H, W), jnp.float32),
        compiler_params=pltpu.CompilerParams(
            dimension_semantics=("parallel",),
            vmem_limit_bytes=44 * 1024 * 1024,
        ),
        name="filter_apply",
    )(x, rm, cm)


# speedup vs baseline: 4.5858x; 4.5858x over previous
import numpy as np
import jax
import jax.numpy as jnp
from jax.experimental import pallas as pl
from jax.experimental.pallas import tpu as pltpu

H, W = 5000, 4000
h, w = 1000, 800
RB = 200
NB = H // RB
ORS = RB // 5
KSLOTS = 48


def _build_row_weights() -> np.ndarray:
    m = np.zeros((KSLOTS, RB), np.float32)
    for k in range(ORS + 2):
        c = 5 * k - 3
        for l in range(max(0, c - 4), min(RB - 1, c + 4) + 1):
            m[k, l] = (1.0 - abs(l - c) / 5.0) / 5.0
    return m


def _build_col_weights() -> np.ndarray:
    m = np.zeros((5, 1024, 160), np.float32)
    for c in range(5):
        for cc in range(160):
            ctr = 5 * (160 * c + cc) + 2
            for j in range(max(0, ctr - 4), min(W - 1, ctr + 4) + 1):
                jj = j - (800 * c - 128)
                m[c, jj, cc] = (1.0 - abs(j - ctr) / 5.0) / 5.0
    return m.reshape(5 * 1024, 160)


_WR = _build_row_weights()
_WC = _build_col_weights()


def _downsample_kernel(x_ref, wr_ref, wc_ref, acc_ref):
    b = pl.program_id(0)

    @pl.when(b == 0)
    def _():
        acc_ref[...] = jnp.zeros_like(acc_ref)

    xb = x_ref[...].astype(jnp.bfloat16)
    t1 = jax.lax.dot_general(
        wr_ref[...], xb, (((1,), (0,)), ((), ())),
        preferred_element_type=jnp.float32)
    t1p = jnp.concatenate(
        [jnp.zeros((KSLOTS, 128), jnp.bfloat16),
         t1.astype(jnp.bfloat16),
         jnp.zeros((KSLOTS, 96), jnp.bfloat16)], axis=1)
    t2 = jnp.concatenate([
        jax.lax.dot_general(
            t1p[:, 800 * c:800 * c + 1024],
            wc_ref[...][1024 * c:1024 * c + 1024, :],
            (((1,), (0,)), ((), ())),
            preferred_element_type=jnp.float32)
        for c in range(5)], axis=1)
    acc_ref[pl.ds(ORS * b, KSLOTS), :] += t2


def _slide13(v, combine, axis):
    n = v.shape[axis] - 24

    def sl(a, start, size):
        idx = [slice(None)] * a.ndim
        idx[axis] = slice(start, start + size)
        return a[tuple(idx)]

    a1 = combine(sl(v, 0, n + 23), sl(v, 1, n + 23))
    a2 = combine(sl(a1, 0, n + 21), sl(a1, 2, n + 21))
    a4 = combine(sl(a2, 0, n + 17), sl(a2, 4, n + 17))
    return combine(combine(sl(a4, 2, n), sl(a2, 10, n)), sl(v, 14, n))


def _morph_open_13(x1):
    zr = jnp.zeros((8, w), jnp.float32)
    zr2 = jnp.zeros((16, w), jnp.float32)
    zc = jnp.zeros((h, 8), jnp.float32)
    zc2 = jnp.zeros((h, 16), jnp.float32)

    fr = jnp.concatenate([zr, x1, zr2], axis=0)
    er_r = _slide13(fr, jnp.minimum, 0)
    fc = jnp.concatenate([zc, er_r, zc2], axis=1)
    er = _slide13(fc, jnp.minimum, 1)

    gr = jnp.concatenate([zr, er, zr2], axis=0)
    di_r = _slide13(gr, jnp.maximum, 0)
    gc = jnp.concatenate([zc, di_r, zc2], axis=1)
    return _slide13(gc, jnp.maximum, 1)


def _mask_kernel(acc_ref, rm_ref, cm_ref):
    x1 = acc_ref[...][1:h + 1, :]
    di = _morph_open_13(x1)
    p = jnp.where(di > 0.0, 1.0, 0.0).astype(jnp.float32)

    y_cnt = jnp.sum(p, axis=1, keepdims=True)
    P = jnp.where(y_cnt > 0.0, 1.0, 0.0).astype(jnp.float32)
    Pprev = jnp.concatenate([P[0:1], P[0:h - 1]], axis=0)
    Pnext = jnp.concatenate([P[1:h], P[h - 1:h]], axis=0)
    m01 = jnp.maximum(Pprev, P)
    m234 = jnp.maximum(P, Pnext)
    lane = jax.lax.broadcasted_iota(jnp.int32, (h, 8), 1)
    rm_ref[...] = jnp.where(lane < 2, m01, m234)

    x_cnt = jnp.sum(p, axis=0, keepdims=True)
    Q = jnp.where(x_cnt > 0.0, 1.0, 0.0).astype(jnp.float32)
    Qprev = jnp.concatenate([Q[:, 0:1], Q[:, 0:w - 1]], axis=1)
    Qnext = jnp.concatenate([Q[:, 1:w], Q[:, w - 1:w]], axis=1)
    n01 = jnp.maximum(Qprev, Q)
    n234 = jnp.maximum(Q, Qnext)
    sub = jax.lax.broadcasted_iota(jnp.int32, (8, w), 0)
    cm_ref[...] = jnp.where(sub < 2, n01, n234)


def _apply_kernel(x_ref, rm_ref, cm_ref, o_ref):
    o_ref[...] = x_ref[...] * rm_ref[...] * cm_ref[...]


def kernel(x):
    wr = jnp.asarray(_WR, dtype=jnp.bfloat16)
    wc = jnp.asarray(_WC, dtype=jnp.bfloat16)

    acc = pl.pallas_call(
        _downsample_kernel,
        grid=(NB,),
        in_specs=[
            pl.BlockSpec((RB, W), lambda b: (b, 0)),
            pl.BlockSpec((KSLOTS, RB), lambda b: (0, 0)),
            pl.BlockSpec((5 * 1024, 160), lambda b: (0, 0)),
        ],
        out_specs=pl.BlockSpec((h + 8, w), lambda b: (0, 0)),
        out_shape=jax.ShapeDtypeStruct((h + 8, w), jnp.float32),
        compiler_params=pltpu.CompilerParams(
            dimension_semantics=("arbitrary",),
            vmem_limit_bytes=48 * 1024 * 1024,
        ),
        name="filter_downsample",
    )(x, wr, wc)

    rm5, cm5 = pl.pallas_call(
        _mask_kernel,
        out_shape=[
            jax.ShapeDtypeStruct((h, 8), jnp.float32),
            jax.ShapeDtypeStruct((8, w), jnp.float32),
        ],
        compiler_params=pltpu.CompilerParams(
            vmem_limit_bytes=52 * 1024 * 1024,
        ),
        name="filter_mask",
    )(acc)

    rm = rm5[:, :5].reshape(H, 1)
    cm = cm5[:5, :].T.reshape(1, W)

    return pl.pallas_call(
        _apply_kernel,
        grid=(NB,),
        in_specs=[
            pl.BlockSpec((RB, W), lambda b: (b, 0)),
            pl.BlockSpec((RB, 1), lambda b: (b, 0)),
            pl.BlockSpec((1, W), lambda b: (0, 0)),
        ],
        out_specs=pl.BlockSpec((RB, W), lambda b: (b, 0)),
        out_shape=jax.ShapeDtypeStruct((H, W), jnp.float32),
        compiler_params=pltpu.CompilerParams(
            dimension_semantics=("parallel",),
            vmem_limit_bytes=44 * 1024 * 1024,
        ),
        name="filter_apply",
    )(x, rm, cm)


# speedup vs baseline: 5.1326x; 1.1192x over previous
import numpy as np
import jax
import jax.numpy as jnp
from jax.experimental import pallas as pl
from jax.experimental.pallas import tpu as pltpu

H, W = 5000, 4000
h, w = 1000, 800
RB = 200
NB = H // RB
ORS = RB // 5
KSLOTS = 48


def _build_row_weights() -> np.ndarray:
    m = np.zeros((KSLOTS, RB), np.float32)
    for k in range(ORS + 2):
        c = 5 * k - 3
        for l in range(max(0, c - 4), min(RB - 1, c + 4) + 1):
            m[k, l] = (1.0 - abs(l - c) / 5.0) / 5.0
    return m


def _build_col_weights() -> np.ndarray:
    m = np.zeros((5, 1024, 160), np.float32)
    for c in range(5):
        for cc in range(160):
            ctr = 5 * (160 * c + cc) + 2
            for j in range(max(0, ctr - 4), min(W - 1, ctr + 4) + 1):
                jj = j - (800 * c - 128)
                m[c, jj, cc] = (1.0 - abs(j - ctr) / 5.0) / 5.0
    return m.reshape(5 * 1024, 160)


_WR = _build_row_weights()
_WC = _build_col_weights()


def _downsample_kernel(x_ref, wr_ref, wc_ref, rm_ref, cm_ref, acc_ref):
    b = pl.program_id(0)

    @pl.when(b == 0)
    def _():
        acc_ref[...] = jnp.zeros_like(acc_ref)

    xb = x_ref[...].astype(jnp.bfloat16)
    t1 = jax.lax.dot_general(
        wr_ref[...], xb, (((1,), (0,)), ((), ())),
        preferred_element_type=jnp.float32)
    t1p = jnp.concatenate(
        [jnp.zeros((KSLOTS, 128), jnp.bfloat16),
         t1.astype(jnp.bfloat16),
         jnp.zeros((KSLOTS, 96), jnp.bfloat16)], axis=1)
    t2 = jnp.concatenate([
        jax.lax.dot_general(
            t1p[:, 800 * c:800 * c + 1024],
            wc_ref[...][1024 * c:1024 * c + 1024, :],
            (((1,), (0,)), ((), ())),
            preferred_element_type=jnp.float32)
        for c in range(5)], axis=1)
    acc_ref[pl.ds(ORS * b, KSLOTS), :] += t2

    @pl.when(b == NB - 1)
    def _():
        _mask_tail(acc_ref, rm_ref, cm_ref)


def _slide13(v, combine, axis):
    n = v.shape[axis] - 24

    def sl(a, start, size):
        idx = [slice(None)] * a.ndim
        idx[axis] = slice(start, start + size)
        return a[tuple(idx)]

    a1 = combine(sl(v, 0, n + 23), sl(v, 1, n + 23))
    a2 = combine(sl(a1, 0, n + 21), sl(a1, 2, n + 21))
    a4 = combine(sl(a2, 0, n + 17), sl(a2, 4, n + 17))
    return combine(combine(sl(a4, 2, n), sl(a2, 10, n)), sl(v, 14, n))


def _morph_open_13(x1):
    dt = x1.dtype
    zr = jnp.zeros((8, w), dt)
    zr2 = jnp.zeros((16, w), dt)
    zc = jnp.zeros((h, 8), dt)
    zc2 = jnp.zeros((h, 16), dt)

    fr = jnp.concatenate([zr, x1, zr2], axis=0)
    er_r = _slide13(fr, jnp.minimum, 0)
    fc = jnp.concatenate([zc, er_r, zc2], axis=1)
    er = _slide13(fc, jnp.minimum, 1)

    gr = jnp.concatenate([zr, er, zr2], axis=0)
    di_r = _slide13(gr, jnp.maximum, 0)
    gc = jnp.concatenate([zc, di_r, zc2], axis=1)
    return _slide13(gc, jnp.maximum, 1)


def _mask_tail(acc_ref, rm_ref, cm_ref):
    x1 = acc_ref[...][1:h + 1, :].astype(jnp.bfloat16)
    di = _morph_open_13(x1)

    y_max = jnp.max(di, axis=1, keepdims=True)
    P = jnp.where(y_max.astype(jnp.float32) > 0.0, 1.0, 0.0).astype(jnp.float32)
    Pprev = jnp.concatenate([P[0:1], P[0:h - 1]], axis=0)
    Pnext = jnp.concatenate([P[1:h], P[h - 1:h]], axis=0)
    m01 = jnp.maximum(Pprev, P)
    m234 = jnp.maximum(P, Pnext)
    lane = jax.lax.broadcasted_iota(jnp.int32, (h, 8), 1)
    rm_ref[...] = jnp.where(lane < 2, m01, m234)

    x_max = jnp.max(di, axis=0, keepdims=True)
    Q = jnp.where(x_max.astype(jnp.float32) > 0.0, 1.0, 0.0).astype(jnp.float32)
    Qprev = jnp.concatenate([Q[:, 0:1], Q[:, 0:w - 1]], axis=1)
    Qnext = jnp.concatenate([Q[:, 1:w], Q[:, w - 1:w]], axis=1)
    n01 = jnp.maximum(Qprev, Q)
    n234 = jnp.maximum(Q, Qnext)
    sub = jax.lax.broadcasted_iota(jnp.int32, (8, w), 0)
    cm_ref[...] = jnp.where(sub < 2, n01, n234)


def _apply_kernel(x_ref, rm_ref, cm_ref, o_ref):
    o_ref[...] = x_ref[...] * rm_ref[...] * cm_ref[...]


def kernel(x):
    wr = jnp.asarray(_WR, dtype=jnp.bfloat16)
    wc = jnp.asarray(_WC, dtype=jnp.bfloat16)

    rm5, cm5 = pl.pallas_call(
        _downsample_kernel,
        grid=(NB,),
        in_specs=[
            pl.BlockSpec((RB, W), lambda b: (b, 0)),
            pl.BlockSpec((KSLOTS, RB), lambda b: (0, 0)),
            pl.BlockSpec((5 * 1024, 160), lambda b: (0, 0)),
        ],
        out_specs=[
            pl.BlockSpec((h, 8), lambda b: (0, 0)),
            pl.BlockSpec((8, w), lambda b: (0, 0)),
        ],
        out_shape=[
            jax.ShapeDtypeStruct((h, 8), jnp.float32),
            jax.ShapeDtypeStruct((8, w), jnp.float32),
        ],
        scratch_shapes=[pltpu.VMEM((h + 8, w), jnp.float32)],
        compiler_params=pltpu.CompilerParams(
            dimension_semantics=("arbitrary",),
            vmem_limit_bytes=52 * 1024 * 1024,
        ),
        name="filter_downsample",
    )(x, wr, wc)

    rm = rm5[:, :5].reshape(H, 1)
    cm = cm5[:5, :].T.reshape(1, W)

    return pl.pallas_call(
        _apply_kernel,
        grid=(NB,),
        in_specs=[
            pl.BlockSpec((RB, W), lambda b: (b, 0)),
            pl.BlockSpec((RB, 1), lambda b: (b, 0)),
            pl.BlockSpec((1, W), lambda b: (0, 0)),
        ],
        out_specs=pl.BlockSpec((RB, W), lambda b: (b, 0)),
        out_shape=jax.ShapeDtypeStruct((H, W), jnp.float32),
        compiler_params=pltpu.CompilerParams(
            dimension_semantics=("parallel",),
            vmem_limit_bytes=44 * 1024 * 1024,
        ),
        name="filter_apply",
    )(x, rm, cm)


# speedup vs baseline: 5.2731x; 1.0274x over previous
import numpy as np
import jax
import jax.numpy as jnp
from jax.experimental import pallas as pl
from jax.experimental.pallas import tpu as pltpu

H, W = 5000, 4000
h, w = 1000, 800
RB = 200
NB = H // RB
ORS = RB // 5
KSLOTS = 48


def _build_row_weights() -> np.ndarray:
    m = np.zeros((KSLOTS, RB), np.float32)
    for k in range(ORS + 2):
        c = 5 * k - 3
        for l in range(max(0, c - 4), min(RB - 1, c + 4) + 1):
            m[k, l] = (1.0 - abs(l - c) / 5.0) / 5.0
    return m


def _build_col_weights() -> np.ndarray:
    m = np.zeros((5, 1024, 160), np.float32)
    for c in range(5):
        for cc in range(160):
            ctr = 5 * (160 * c + cc) + 2
            for j in range(max(0, ctr - 4), min(W - 1, ctr + 4) + 1):
                jj = j - (800 * c - 128)
                m[c, jj, cc] = (1.0 - abs(j - ctr) / 5.0) / 5.0
    return m.reshape(5 * 1024, 160)


_WR = _build_row_weights()
_WC = _build_col_weights()


def _downsample_kernel(x_ref, wr_ref, wc_ref, rm_ref, cm_ref, acc_ref):
    b = pl.program_id(0)

    @pl.when(b == 0)
    def _():
        acc_ref[...] = jnp.zeros_like(acc_ref)

    xb = x_ref[...].astype(jnp.bfloat16)
    t1 = jax.lax.dot_general(
        wr_ref[...], xb, (((1,), (0,)), ((), ())),
        preferred_element_type=jnp.float32)
    t1p = jnp.concatenate(
        [jnp.zeros((KSLOTS, 128), jnp.bfloat16),
         t1.astype(jnp.bfloat16),
         jnp.zeros((KSLOTS, 96), jnp.bfloat16)], axis=1)
    t2 = jnp.concatenate([
        jax.lax.dot_general(
            t1p[:, 800 * c:800 * c + 1024],
            wc_ref[...][1024 * c:1024 * c + 1024, :],
            (((1,), (0,)), ((), ())),
            preferred_element_type=jnp.float32)
        for c in range(5)], axis=1)
    acc_ref[pl.ds(ORS * b, KSLOTS), :] += t2

    @pl.when(b == NB - 1)
    def _():
        _mask_tail(acc_ref, rm_ref, cm_ref)


def _slide13(v, combine, axis):
    n = v.shape[axis] - 24

    def sl(a, start, size):
        idx = [slice(None)] * a.ndim
        idx[axis] = slice(start, start + size)
        return a[tuple(idx)]

    a1 = combine(sl(v, 0, n + 23), sl(v, 1, n + 23))
    a2 = combine(sl(a1, 0, n + 21), sl(a1, 2, n + 21))
    a4 = combine(sl(a2, 0, n + 17), sl(a2, 4, n + 17))
    return combine(combine(sl(a4, 2, n), sl(a2, 10, n)), sl(v, 14, n))


def _morph_open_13(x1):
    dt = x1.dtype
    zr = jnp.zeros((8, w), dt)
    zr2 = jnp.zeros((16, w), dt)
    zc = jnp.zeros((h, 8), dt)
    zc2 = jnp.zeros((h, 16), dt)

    fr = jnp.concatenate([zr, x1, zr2], axis=0)
    er_r = _slide13(fr, jnp.minimum, 0)
    fc = jnp.concatenate([zc, er_r, zc2], axis=1)
    er = _slide13(fc, jnp.minimum, 1)

    gr = jnp.concatenate([zr, er, zr2], axis=0)
    di_r = _slide13(gr, jnp.maximum, 0)
    gc = jnp.concatenate([zc, di_r, zc2], axis=1)
    return _slide13(gc, jnp.maximum, 1)


def _mask_tail(acc_ref, rm_ref, cm_ref):
    x1 = acc_ref[...][1:h + 1, :].astype(jnp.bfloat16)
    di = _morph_open_13(x1)

    y_max = jnp.max(di, axis=1, keepdims=True)
    P = jnp.where(y_max.astype(jnp.float32) > 0.0, 1.0, 0.0).astype(jnp.float32)
    Pprev = jnp.concatenate([P[0:1], P[0:h - 1]], axis=0)
    Pnext = jnp.concatenate([P[1:h], P[h - 1:h]], axis=0)
    m01 = jnp.maximum(Pprev, P)
    m234 = jnp.maximum(P, Pnext)
    lane = jax.lax.broadcasted_iota(jnp.int32, (h, 8), 1)
    rm_ref[...] = jnp.where(lane < 2, m01, m234)

    x_max = jnp.max(di, axis=0, keepdims=True)
    Q = jnp.where(x_max.astype(jnp.float32) > 0.0, 1.0, 0.0).astype(jnp.float32)
    Qprev = jnp.concatenate([Q[:, 0:1], Q[:, 0:w - 1]], axis=1)
    Qnext = jnp.concatenate([Q[:, 1:w], Q[:, w - 1:w]], axis=1)
    n01 = jnp.maximum(Qprev, Q)
    n234 = jnp.maximum(Q, Qnext)
    sub = jax.lax.broadcasted_iota(jnp.int32, (8, w), 0)
    cm_ref[...] = jnp.where(sub < 2, n01, n234)


def _apply_kernel(x_ref, rm_ref, cm_ref, o_ref):
    o_ref[...] = x_ref[...] * rm_ref[...] * cm_ref[...]


def kernel(x):
    wr = jnp.asarray(_WR, dtype=jnp.bfloat16)
    wc = jnp.asarray(_WC, dtype=jnp.bfloat16)

    rm5, cm5 = pl.pallas_call(
        _downsample_kernel,
        grid=(NB,),
        in_specs=[
            pl.BlockSpec((RB, W), lambda b: (b, 0)),
            pl.BlockSpec((KSLOTS, RB), lambda b: (0, 0)),
            pl.BlockSpec((5 * 1024, 160), lambda b: (0, 0)),
        ],
        out_specs=[
            pl.BlockSpec((h, 8), lambda b: (0, 0)),
            pl.BlockSpec((8, w), lambda b: (0, 0)),
        ],
        out_shape=[
            jax.ShapeDtypeStruct((h, 8), jnp.float32),
            jax.ShapeDtypeStruct((8, w), jnp.float32),
        ],
        scratch_shapes=[pltpu.VMEM((h + 8, w), jnp.float32)],
        compiler_params=pltpu.CompilerParams(
            dimension_semantics=("arbitrary",),
            vmem_limit_bytes=52 * 1024 * 1024,
        ),
        name="filter_downsample",
    )(x, wr, wc)

    rm = rm5[:, :5].reshape(H, 1)
    cm = cm5[:5, :].T.reshape(1, W)

    return pl.pallas_call(
        _apply_kernel,
        grid=(5, 2),
        in_specs=[
            pl.BlockSpec((1000, 2048), lambda i, j: (i, j)),
            pl.BlockSpec((1000, 1), lambda i, j: (i, 0)),
            pl.BlockSpec((1, 2048), lambda i, j: (0, j)),
        ],
        out_specs=pl.BlockSpec((1000, 2048), lambda i, j: (i, j)),
        out_shape=jax.ShapeDtypeStruct((H, W), jnp.float32),
        compiler_params=pltpu.CompilerParams(
            dimension_semantics=("parallel", "parallel"),
            vmem_limit_bytes=44 * 1024 * 1024,
        ),
        name="filter_apply",
    )(x, rm, cm)
